# baseline (speedup 1.0000x reference)
import functools
import jax
import jax.numpy as jnp
from jax import lax
from jax.experimental import pallas as pl
from jax.experimental.pallas import tpu as pltpu
from jax.experimental.pallas import tpu_sc as plsc

EMBED = 2048
NLEV = 16
L = 16
NW = 32
B0 = 1024
B1 = 20
N = B0 * B1
BPW = B0 // NW
CHUNK = EMBED // L
UNR = 32
TBL = (NLEV - 1) * EMBED


def _encode_body(w_ref, t_ref, u_ref):
    w = w_ref[...]
    t = t_ref[...]
    ws = w[:-1, :]
    we = w[1:, :]
    u_ref[...] = ws * jnp.where(ws == we, -1.5, t)


def _encode(weight, threshold):
    return pl.pallas_call(
        _encode_body,
        out_shape=jax.ShapeDtypeStruct((NLEV - 1, EMBED), jnp.float32),
    )(weight, threshold)


def _sc_body(x_hbm, u_hbm, out_hbm, x_v, u_v, tau_v, gb_v, buf_v, sem0, sem1):
    cid = lax.axis_index("c")
    sid = lax.axis_index("s")
    wid = sid * 2 + cid
    base_b = wid * BPW
    pltpu.sync_copy(u_hbm, u_v)
    pltpu.sync_copy(x_hbm.at[pl.ds(base_b * B1, BPW * B1)], x_v)

    def pre(j, carry):
        xv = x_v[pl.ds(j * L, L)]
        span = jnp.clip(xv * float(NLEV - 1), 0.0, float(NLEV - 1))
        idxi = jnp.minimum(span.astype(jnp.int32), NLEV - 2)
        tau_v[pl.ds(j * L, L)] = span - idxi.astype(jnp.float32)
        gb_v[pl.ds(j * L, L)] = idxi * EMBED
        return carry

    lax.fori_loop(0, (BPW * B1) // L, pre, 0)

    def batch(g, carry):
        j = lax.div(g, 2)
        h = lax.rem(g, 2)
        slot = lax.rem(g, 2)
        @pl.when(g >= 2)
        def _wait():
            @pl.when(slot == 0)
            def _w0():
                pltpu.make_async_copy(buf_v.at[0], out_hbm.at[0, pl.ds(base_b, L)], sem0).wait()

            @pl.when(slot == 1)
            def _w1():
                pltpu.make_async_copy(buf_v.at[1], out_hbm.at[0, pl.ds(base_b, L)], sem1).wait()

        def row(r, carry2):
            rowidx = (h * L + r) * B1 + j
            tau = jnp.full((L,), tau_v[pl.ds(rowidx, L)][0], jnp.float32)
            gb = gb_v[pl.ds(rowidx, L)][0]
            taub = plsc.bitcast(tau, jnp.int32)

            def col(jc, sb):
                us = [u_v[pl.ds(sb + k * L, L)] for k in range(UNR)]
                for k in range(UNR):
                    d = plsc.bitcast(us[k], jnp.int32) - taub
                    val = jnp.where(d < 0, 1.0, -1.0).astype(jnp.float32)
                    buf_v[slot, r, pl.ds(jc * (UNR * L) + k * L, L)] = val
                return sb + UNR * L

            lax.fori_loop(0, CHUNK // UNR, col, gb)
            return carry2

        lax.fori_loop(0, L, row, 0)
        dst_row = base_b + h * L

        @pl.when(slot == 0)
        def _s0():
            pltpu.async_copy(buf_v.at[0], out_hbm.at[j, pl.ds(dst_row, L)], sem0)

        @pl.when(slot == 1)
        def _s1():
            pltpu.async_copy(buf_v.at[1], out_hbm.at[j, pl.ds(dst_row, L)], sem1)

        return carry

    lax.fori_loop(0, 2 * B1, batch, 0)
    pltpu.make_async_copy(buf_v.at[0], out_hbm.at[0, pl.ds(base_b, L)], sem0).wait()
    pltpu.make_async_copy(buf_v.at[1], out_hbm.at[0, pl.ds(base_b, L)], sem1).wait()


@jax.jit
def _run(x_flat, u_flat):
    mesh = plsc.VectorSubcoreMesh(core_axis_name="c", subcore_axis_name="s")
    sc = pl.kernel(
        _sc_body,
        out_type=jax.ShapeDtypeStruct((B1, B0, EMBED), jnp.float32),
        mesh=mesh,
        compiler_params=pltpu.CompilerParams(
            needs_layout_passes=False,
            use_tc_tiling_on_sc=True,
        ),
        scratch_types=[
            pltpu.VMEM((BPW * B1,), jnp.float32),
            pltpu.VMEM((TBL,), jnp.float32),
            pltpu.VMEM((BPW * B1 + L,), jnp.float32),
            pltpu.VMEM((BPW * B1 + L,), jnp.int32),
            pltpu.VMEM((2, L, EMBED), jnp.float32),
            pltpu.SemaphoreType.DMA,
            pltpu.SemaphoreType.DMA,
        ],
    )
    return sc(x_flat, u_flat)


def kernel(input, weight, threshold):
    u = _encode(weight, threshold)
    out_t = _run(input.reshape(N), u.reshape(TBL))
    return jnp.transpose(out_t, (1, 0, 2)).reshape(*input.shape, EMBED)


# speedup vs baseline: 1.0101x; 1.0101x over previous
import functools
import jax
import jax.numpy as jnp
from jax import lax
from jax.experimental import pallas as pl
from jax.experimental.pallas import tpu as pltpu
from jax.experimental.pallas import tpu_sc as plsc

EMBED = 2048
NLEV = 16
L = 16
NW = 32
B0 = 1024
B1 = 20
N = B0 * B1
BPW = B0 // NW
CHUNK = EMBED // L
UNR = 16
TBL = (NLEV - 1) * EMBED


def _encode_body(w_ref, t_ref, u_ref):
    w = w_ref[...]
    t = t_ref[...]
    ws = w[:-1, :]
    we = w[1:, :]
    u_ref[...] = ws * jnp.where(ws == we, -1.5, t)


def _encode(weight, threshold):
    return pl.pallas_call(
        _encode_body,
        out_shape=jax.ShapeDtypeStruct((NLEV - 1, EMBED), jnp.float32),
    )(weight, threshold)


def _sc_body(x_hbm, u_hbm, out_hbm, x_v, u_v, tau_v, gb_v, buf_v, sem0, sem1):
    cid = lax.axis_index("c")
    sid = lax.axis_index("s")
    wid = sid * 2 + cid
    base_b = wid * BPW
    pltpu.sync_copy(u_hbm, u_v)
    pltpu.sync_copy(x_hbm.at[pl.ds(base_b * B1, BPW * B1)], x_v)

    def pre(j, carry):
        xv = x_v[pl.ds(j * L, L)]
        span = jnp.clip(xv * float(NLEV - 1), 0.0, float(NLEV - 1))
        idxi = jnp.minimum(span.astype(jnp.int32), NLEV - 2)
        tau_v[pl.ds(j * L, L)] = span - idxi.astype(jnp.float32)
        gb_v[pl.ds(j * L, L)] = idxi * EMBED
        return carry

    lax.fori_loop(0, (BPW * B1) // L, pre, 0)

    def batch(g, carry):
        j = lax.div(g, 2)
        h = lax.rem(g, 2)
        slot = lax.rem(g, 2)
        @pl.when(g >= 2)
        def _wait():
            @pl.when(slot == 0)
            def _w0():
                pltpu.make_async_copy(buf_v.at[0], out_hbm.at[0, pl.ds(base_b, L)], sem0).wait()

            @pl.when(slot == 1)
            def _w1():
                pltpu.make_async_copy(buf_v.at[1], out_hbm.at[0, pl.ds(base_b, L)], sem1).wait()

        def row(r, carry2):
            rowidx = (h * L + r) * B1 + j
            tau = jnp.full((L,), tau_v[pl.ds(rowidx, L)][0], jnp.float32)
            gb = gb_v[pl.ds(rowidx, L)][0]
            taub = plsc.bitcast(tau, jnp.int32)

            def col(jc, sb):
                us = [u_v[pl.ds(sb + k * L, L)] for k in range(UNR)]
                for k in range(UNR):
                    d = plsc.bitcast(us[k], jnp.int32) - taub
                    val = jnp.where(d < 0, 1.0, -1.0).astype(jnp.float32)
                    buf_v[slot, r, pl.ds(jc * (UNR * L) + k * L, L)] = val
                return sb + UNR * L

            lax.fori_loop(0, CHUNK // UNR, col, gb)
            return carry2

        lax.fori_loop(0, L, row, 0)
        dst_row = base_b + h * L

        @pl.when(slot == 0)
        def _s0():
            pltpu.async_copy(buf_v.at[0], out_hbm.at[j, pl.ds(dst_row, L)], sem0)

        @pl.when(slot == 1)
        def _s1():
            pltpu.async_copy(buf_v.at[1], out_hbm.at[j, pl.ds(dst_row, L)], sem1)

        return carry

    lax.fori_loop(0, 2 * B1, batch, 0)
    pltpu.make_async_copy(buf_v.at[0], out_hbm.at[0, pl.ds(base_b, L)], sem0).wait()
    pltpu.make_async_copy(buf_v.at[1], out_hbm.at[0, pl.ds(base_b, L)], sem1).wait()


@jax.jit
def _run(x_flat, u_flat):
    mesh = plsc.VectorSubcoreMesh(core_axis_name="c", subcore_axis_name="s")
    sc = pl.kernel(
        _sc_body,
        out_type=jax.ShapeDtypeStruct((B1, B0, EMBED), jnp.float32),
        mesh=mesh,
        compiler_params=pltpu.CompilerParams(
            needs_layout_passes=False,
            use_tc_tiling_on_sc=True,
        ),
        scratch_types=[
            pltpu.VMEM((BPW * B1,), jnp.float32),
            pltpu.VMEM((TBL,), jnp.float32),
            pltpu.VMEM((BPW * B1 + L,), jnp.float32),
            pltpu.VMEM((BPW * B1 + L,), jnp.int32),
            pltpu.VMEM((2, L, EMBED), jnp.float32),
            pltpu.SemaphoreType.DMA,
            pltpu.SemaphoreType.DMA,
        ],
    )
    return sc(x_flat, u_flat)


def kernel(input, weight, threshold):
    u = _encode(weight, threshold)
    out_t = _run(input.reshape(N), u.reshape(TBL))
    return jnp.transpose(out_t, (1, 0, 2)).reshape(*input.shape, EMBED)
